# Initial kernel scaffold; baseline (speedup 1.0000x reference)
import jax
import jax.numpy as jnp
from jax.experimental import pallas as pl


def kernel(x, edge_index, adj_vals, W, b):
    raise NotImplementedError("write your pallas kernel here")


# speedup vs baseline: 5.9301x; 5.9301x over previous
import functools

import jax
import jax.numpy as jnp
from jax import lax
from jax.experimental import pallas as pl
from jax.experimental.pallas import tpu as pltpu
from jax.experimental.pallas import tpu_sc as plsc

N = 10000
E = 320000
D = 128

NC = 2
NS = 16
NW = NC * NS
E_PER_TILE = E // NW
CHUNK = 80
N_CHUNKS = E_PER_TILE // CHUNK
NPAD = 10240
ROWS_PER_TILE = NPAD // NS
ZROWS = 64
BLK = 5
N_BLKS = N_CHUNKS // BLK


def _mm_body(x_ref, w_ref, b_ref, o_ref):
    o_ref[...] = (
        jnp.dot(x_ref[...], w_ref[...], preferred_element_type=jnp.float32)
        + b_ref[...]
    )


def _matmul(x, W, b):
    BM = 2000
    return pl.pallas_call(
        _mm_body,
        grid=(N // BM,),
        in_specs=[
            pl.BlockSpec((BM, D), lambda i: (i, 0)),
            pl.BlockSpec((D, D), lambda i: (0, 0)),
            pl.BlockSpec((1, D), lambda i: (0, 0)),
        ],
        out_specs=pl.BlockSpec((BM, D), lambda i: (i, 0)),
        out_shape=jax.ShapeDtypeStruct((N, D), jnp.float32),
    )(x, W, b.reshape(1, D))


def _comb_body(p_ref, o_ref):
    o_ref[...] = p_ref[0] + p_ref[1]


def _combine(partial):
    BM = 2000
    return pl.pallas_call(
        _comb_body,
        grid=(N // BM,),
        in_specs=[pl.BlockSpec((NC, BM, D), lambda i: (0, i, 0))],
        out_specs=pl.BlockSpec((BM, D), lambda i: (i, 0)),
        out_shape=jax.ShapeDtypeStruct((N, D), jnp.float32),
    )(partial)


_MESH = plsc.VectorSubcoreMesh(
    core_axis_name="c", subcore_axis_name="s", num_cores=NC, num_subcores=NS
)


@functools.partial(
    pl.kernel,
    out_type=jax.ShapeDtypeStruct((NC, NPAD, D), jnp.float32),
    mesh=_MESH,
    scratch_types=[
        pltpu.VMEM((BLK, CHUNK), jnp.int32),
        pltpu.VMEM((BLK, CHUNK), jnp.int32),
        pltpu.VMEM((BLK, CHUNK), jnp.float32),
        pltpu.VMEM((CHUNK, D), jnp.float32),
        pltpu.VMEM((ZROWS, D), jnp.float32),
        pltpu.VMEM_SHARED((NPAD, D), jnp.float32),
        pltpu.SemaphoreType.DMA,
    ],
)
def _sc_scatter(h_hbm, col_hbm, row_hbm, val_hbm, out_hbm,
                col_v, row_v, val_v, rows_v, zero_v, acc_sh, sem):
    c = lax.axis_index("c")
    s = lax.axis_index("s")
    wid = c * NS + s

    zvec = jnp.zeros((16,), jnp.float32)

    def _zero_row(i, carry):
        for j in range(D // 16):
            zero_v[i, pl.ds(j * 16, 16)] = zvec
        return carry

    lax.fori_loop(0, ZROWS, _zero_row, 0)
    for t in range(ROWS_PER_TILE // ZROWS):
        pltpu.sync_copy(
            zero_v, acc_sh.at[pl.ds(s * ROWS_PER_TILE + t * ZROWS, ZROWS)]
        )

    plsc.subcore_barrier()

    def _block(blk, carry):
        pltpu.sync_copy(col_hbm.at[wid, blk], col_v)
        pltpu.sync_copy(row_hbm.at[wid, blk], row_v)
        pltpu.sync_copy(val_hbm.at[wid, blk], val_v)

        def _chunk(k, carry1):
            pltpu.async_copy(h_hbm.at[col_v.at[k]], rows_v, sem).wait()

            def _scale(g, carry2):
                vv = val_v[k, pl.ds(g * 16, 16)]
                for ii in range(16):
                    v = vv[ii]
                    i = g * 16 + ii
                    for j in range(D // 16):
                        sl = pl.ds(j * 16, 16)
                        rows_v[i, sl] = rows_v[i, sl] * v
                return carry2

            lax.fori_loop(0, CHUNK // 16, _scale, 0)

            pltpu.sync_copy(rows_v, acc_sh.at[row_v.at[k]], add=True)
            return carry1

        lax.fori_loop(0, BLK, _chunk, 0)
        return carry

    lax.fori_loop(0, N_BLKS, _block, 0)

    plsc.subcore_barrier()

    rbase = s * ROWS_PER_TILE
    pltpu.sync_copy(
        acc_sh.at[pl.ds(rbase, ROWS_PER_TILE)],
        out_hbm.at[c, pl.ds(rbase, ROWS_PER_TILE)],
    )


def kernel(x, edge_index, adj_vals, W, b):
    h = _matmul(x, W, b)
    col = edge_index[1].reshape(NW, N_BLKS, BLK, CHUNK)
    row = edge_index[0].reshape(NW, N_BLKS, BLK, CHUNK)
    val = adj_vals.reshape(NW, N_BLKS, BLK, CHUNK)
    partial = _sc_scatter(h, col, row, val)
    return _combine(partial)
